# baseline (speedup 1.0000x reference)
import functools

import jax
import jax.numpy as jnp
from jax import lax
from jax.experimental import pallas as pl
from jax.experimental.pallas import tpu as pltpu
from jax.experimental.pallas import tpu_sc as plsc

N = 5000
C = 201
NCLS = 200
C_PAD = 256
N_PAD = 5120
TOPK = 100
K_PAD = 112
NEG = -1e30
NMS_THRESH = 0.5
TI = 256
TJ = 1280
RB = 400


def _score_kernel(logits_ref, score_ref, label_ref):
    l = logits_ref[...]
    col = lax.broadcasted_iota(jnp.int32, l.shape, 1)
    m_all = jnp.max(l, axis=1, keepdims=True)
    z = jnp.sum(jnp.exp(l - m_all), axis=1, keepdims=True)
    lm = jnp.where(col < NCLS, l, NEG)
    m200 = jnp.max(lm, axis=1, keepdims=True)
    score_ref[...] = jnp.exp(m200 - m_all) / z
    is_max = (lm == m200) & (col < NCLS)
    label_ref[...] = jnp.min(jnp.where(is_max, col, C_PAD), axis=1, keepdims=True)


def _nms_kernel(c_col_ref, w_col_ref, s_col_ref, c_row_ref, w_row_ref, s_row_ref,
                key_ref):
    ci = c_col_ref[...]
    wi = w_col_ref[...]
    si = s_col_ref[...]
    st_i = ci - wi / 2.0
    en_i = ci + wi / 2.0
    ln_i = jnp.maximum(en_i - st_i, 0.0)
    ig = pl.program_id(1) * TI + lax.broadcasted_iota(jnp.int32, (1, TI, 1), 1)

    def jstep(j, supp):
        cj = c_row_ref[:, :, pl.ds(j * TJ, TJ)]
        wj = w_row_ref[:, :, pl.ds(j * TJ, TJ)]
        sj = s_row_ref[:, :, pl.ds(j * TJ, TJ)]
        st_j = cj - wj / 2.0
        en_j = cj + wj / 2.0
        ln_j = jnp.maximum(en_j - st_j, 0.0)
        jg = j * TJ + lax.broadcasted_iota(jnp.int32, (1, 1, TJ), 2)
        inter = jnp.maximum(0.0, jnp.minimum(en_i, en_j) - jnp.maximum(st_i, st_j))
        union = (ln_i + ln_j) - inter
        iou = inter / jnp.maximum(union, 1e-8)
        dom = (sj > si) | ((sj == si) & (jg < ig))
        hit = dom & (iou > NMS_THRESH)
        return jnp.maximum(supp, jnp.max(jnp.where(hit, 1.0, 0.0),
                                         axis=2, keepdims=True))

    supp = lax.fori_loop(0, N_PAD // TJ, jstep,
                         jnp.zeros((1, TI, 1), jnp.float32))
    sbits = lax.bitcast_convert_type(si, jnp.int32)
    key = sbits + jnp.where(supp > 0.0, 0, 2 ** 30)
    key_ref[...] = jnp.where(ig < N, key, -2 ** 31)


@functools.cache
def _make_sc_select():
    mesh = plsc.VectorSubcoreMesh(core_axis_name="c", subcore_axis_name="s")

    @functools.partial(
        pl.kernel,
        out_type=[
            jax.ShapeDtypeStruct((2, K_PAD), jnp.float32),
            jax.ShapeDtypeStruct((2, K_PAD), jnp.int32),
            jax.ShapeDtypeStruct((2, K_PAD), jnp.float32),
            jax.ShapeDtypeStruct((2, K_PAD), jnp.float32),
        ],
        mesh=mesh,
        compiler_params=pltpu.CompilerParams(needs_layout_passes=False),
        scratch_types=[
            pltpu.VMEM((N_PAD,), jnp.int32),
            pltpu.VMEM((N_PAD,), jnp.float32),
            pltpu.VMEM((N_PAD,), jnp.int32),
            pltpu.VMEM((N_PAD,), jnp.float32),
            pltpu.VMEM((N_PAD,), jnp.float32),
            pltpu.VMEM((K_PAD,), jnp.float32),
            pltpu.VMEM((K_PAD,), jnp.int32),
            pltpu.VMEM((K_PAD,), jnp.float32),
            pltpu.VMEM((K_PAD,), jnp.float32),
            pltpu.VMEM((16,), jnp.int32),
            pltpu.VMEM((16,), jnp.int32),
        ],
    )
    def sc_select(key_hbm, score_hbm, label_hbm, c_hbm, w_hbm,
                  os_hbm, ol_hbm, oc_hbm, ow_hbm,
                  key_v, score_v, label_v, c_v, w_v, os_v, ol_v, oc_v, ow_v,
                  red_k_v, red_i_v):
        cid = lax.axis_index("c")
        sid = lax.axis_index("s")

        @pl.when(sid == 0)
        def _():
            b = cid
            pltpu.sync_copy(key_hbm.at[b], key_v)
            pltpu.sync_copy(score_hbm.at[b], score_v)
            pltpu.sync_copy(label_hbm.at[b], label_v)
            pltpu.sync_copy(c_hbm.at[b], c_v)
            pltpu.sync_copy(w_hbm.at[b], w_v)
            lane = lax.iota(jnp.int32, 16)
            for g in range(K_PAD // 16):
                os_v[pl.ds(g * 16, 16)] = jnp.zeros((16,), jnp.float32)
                ol_v[pl.ds(g * 16, 16)] = jnp.zeros((16,), jnp.int32)
                oc_v[pl.ds(g * 16, 16)] = jnp.zeros((16,), jnp.float32)
                ow_v[pl.ds(g * 16, 16)] = jnp.zeros((16,), jnp.float32)

            def pick(t, carry):
                def chunk(j, mi_pair):
                    mv, mi = mi_pair
                    v = key_v[pl.ds(j * 16, 16)]
                    idx = j * 16 + lane
                    upd = v > mv
                    return (jnp.where(upd, v, mv), jnp.where(upd, idx, mi))

                mv, mi = lax.fori_loop(
                    0, N_PAD // 16, chunk,
                    (jnp.full((16,), -2 ** 31, jnp.int32), lane))
                for sh in (8, 4, 2, 1):
                    red_k_v[...] = mv
                    red_i_v[...] = mi
                    perm = lane ^ sh
                    k2 = plsc.load_gather(red_k_v, [perm])
                    i2 = plsc.load_gather(red_i_v, [perm])
                    better = (k2 > mv) | ((k2 == mv) & (i2 < mi))
                    mv = jnp.where(better, k2, mv)
                    mi = jnp.where(better, i2, mi)
                widx_v = mi
                tmask = lane == 0
                tpos = jnp.full((16,), t, jnp.int32)
                plsc.store_scatter(key_v, [widx_v],
                                   jnp.full((16,), -2 ** 31, jnp.int32), mask=tmask)
                sv = plsc.load_gather(score_v, [widx_v])
                lv = plsc.load_gather(label_v, [widx_v])
                cv = plsc.load_gather(c_v, [widx_v])
                wv = plsc.load_gather(w_v, [widx_v])
                plsc.store_scatter(os_v, [tpos], sv, mask=tmask)
                plsc.store_scatter(ol_v, [tpos], lv, mask=tmask)
                plsc.store_scatter(oc_v, [tpos], cv, mask=tmask)
                plsc.store_scatter(ow_v, [tpos], wv, mask=tmask)
                return carry

            lax.fori_loop(0, TOPK, pick, 0)
            pltpu.sync_copy(os_v, os_hbm.at[b])
            pltpu.sync_copy(ol_v, ol_hbm.at[b])
            pltpu.sync_copy(oc_v, oc_hbm.at[b])
            pltpu.sync_copy(ow_v, ow_hbm.at[b])

    return sc_select


def _scores_labels(pred_logits):
    B = pred_logits.shape[0]
    logits = jnp.pad(pred_logits.reshape(B * N, C), ((0, 0), (0, C_PAD - C)),
                     constant_values=NEG)
    score_col, label_col = pl.pallas_call(
        _score_kernel,
        grid=(B * N // RB,),
        in_specs=[pl.BlockSpec((RB, C_PAD), lambda r: (r, 0))],
        out_specs=[pl.BlockSpec((RB, 1), lambda r: (r, 0)),
                   pl.BlockSpec((RB, 1), lambda r: (r, 0))],
        out_shape=[jax.ShapeDtypeStruct((B * N, 1), jnp.float32),
                   jax.ShapeDtypeStruct((B * N, 1), jnp.int32)],
    )(logits)
    return score_col.reshape(B, N), label_col.reshape(B, N)


def _nms_keys(c_p, w_p, s_p):
    B = c_p.shape[0]
    col = lambda x: x[..., None]
    row = lambda x: x[:, None, :]
    key = pl.pallas_call(
        _nms_kernel,
        grid=(B, N_PAD // TI),
        in_specs=[
            pl.BlockSpec((1, TI, 1), lambda b, i: (b, i, 0)),
            pl.BlockSpec((1, TI, 1), lambda b, i: (b, i, 0)),
            pl.BlockSpec((1, TI, 1), lambda b, i: (b, i, 0)),
            pl.BlockSpec((1, 1, N_PAD), lambda b, i: (b, 0, 0)),
            pl.BlockSpec((1, 1, N_PAD), lambda b, i: (b, 0, 0)),
            pl.BlockSpec((1, 1, N_PAD), lambda b, i: (b, 0, 0)),
        ],
        out_specs=pl.BlockSpec((1, TI, 1), lambda b, i: (b, i, 0)),
        out_shape=jax.ShapeDtypeStruct((B, N_PAD, 1), jnp.int32),
    )(col(c_p), col(w_p), col(s_p), row(c_p), row(w_p), row(s_p))
    return key.reshape(B, N_PAD)


def kernel(pred_logits, pred_segments, target_lengths):
    scores, labels = _scores_labels(pred_logits)
    seg = pred_segments * target_lengths[:, None, :]
    c = seg[..., 0]
    w = seg[..., 1]
    padr = lambda x, v: jnp.pad(x, ((0, 0), (0, N_PAD - N)), constant_values=v)
    c_p = padr(c, 0.0)
    w_p = padr(w, 0.0)
    s_p = padr(scores, NEG)
    key = _nms_keys(c_p, w_p, s_p)
    label_p = padr(labels, 0)
    os_, ol_, oc_, ow_ = _make_sc_select()(key, s_p, label_p, c_p, w_p)
    out_scores = os_[:, :TOPK]
    out_labels = ol_[:, :TOPK]
    out_segments = jnp.stack([oc_[:, :TOPK], ow_[:, :TOPK]], axis=-1)
    return out_scores, out_labels, out_segments


# speedup vs baseline: 1.4586x; 1.4586x over previous
import functools

import jax
import jax.numpy as jnp
from jax import lax
from jax.experimental import pallas as pl
from jax.experimental.pallas import tpu as pltpu
from jax.experimental.pallas import tpu_sc as plsc

N = 5000
C = 201
NCLS = 200
C_PAD = 256
N_PAD = 5120
TOPK = 100
K_PAD = 112
NEG = -1e30
NMS_THRESH = 0.5
TI = 256
TJ = 1280
RB = 400


def _score_kernel(logits_ref, score_ref, label_ref):
    l = logits_ref[...]
    col = lax.broadcasted_iota(jnp.int32, l.shape, 1)
    m_all = jnp.max(l, axis=1, keepdims=True)
    z = jnp.sum(jnp.exp(l - m_all), axis=1, keepdims=True)
    lm = jnp.where(col < NCLS, l, NEG)
    m200 = jnp.max(lm, axis=1, keepdims=True)
    score_ref[...] = jnp.exp(m200 - m_all) / z
    is_max = (lm == m200) & (col < NCLS)
    label_ref[...] = jnp.min(jnp.where(is_max, col, C_PAD), axis=1, keepdims=True)


def _nms_kernel(c_col_ref, w_col_ref, s_col_ref, c_row_ref, w_row_ref, s_row_ref,
                key_ref):
    ci = c_col_ref[...]
    wi = w_col_ref[...]
    si = s_col_ref[...]
    st_i = ci - wi / 2.0
    en_i = ci + wi / 2.0
    ln_i = jnp.maximum(en_i - st_i, 0.0)
    ig = pl.program_id(1) * TI + lax.broadcasted_iota(jnp.int32, (1, TI, 1), 1)

    def jstep(j, supp):
        cj = c_row_ref[:, :, pl.ds(j * TJ, TJ)]
        wj = w_row_ref[:, :, pl.ds(j * TJ, TJ)]
        sj = s_row_ref[:, :, pl.ds(j * TJ, TJ)]
        st_j = cj - wj / 2.0
        en_j = cj + wj / 2.0
        ln_j = jnp.maximum(en_j - st_j, 0.0)
        jg = j * TJ + lax.broadcasted_iota(jnp.int32, (1, 1, TJ), 2)
        inter = jnp.maximum(0.0, jnp.minimum(en_i, en_j) - jnp.maximum(st_i, st_j))
        union = (ln_i + ln_j) - inter
        iou = inter / jnp.maximum(union, 1e-8)
        dom = (sj > si) | ((sj == si) & (jg < ig))
        hit = dom & (iou > NMS_THRESH)
        return jnp.maximum(supp, jnp.max(jnp.where(hit, 1.0, 0.0),
                                         axis=2, keepdims=True))

    supp = lax.fori_loop(0, N_PAD // TJ, jstep,
                         jnp.zeros((1, TI, 1), jnp.float32))
    sbits = lax.bitcast_convert_type(si, jnp.int32)
    key = sbits + jnp.where(supp > 0.0, 0, 2 ** 30)
    key_ref[...] = jnp.where(ig < N, key, -2 ** 31)


NGRP = N_PAD // 256


@functools.cache
def _make_sc_select():
    mesh = plsc.VectorSubcoreMesh(core_axis_name="c", subcore_axis_name="s")

    @functools.partial(
        pl.kernel,
        out_type=jax.ShapeDtypeStruct((2 * 4 * K_PAD,), jnp.int32),
        mesh=mesh,
        compiler_params=pltpu.CompilerParams(needs_layout_passes=False),
        scratch_types=[
            pltpu.VMEM((N_PAD,), jnp.int32),
            pltpu.VMEM((N_PAD,), jnp.int32),
            pltpu.VMEM((N_PAD,), jnp.int32),
            pltpu.VMEM((N_PAD,), jnp.int32),
            pltpu.VMEM((N_PAD,), jnp.int32),
            pltpu.VMEM((4 * K_PAD,), jnp.int32),
            pltpu.VMEM((16 * NGRP,), jnp.int32),
            pltpu.VMEM((16 * NGRP,), jnp.int32),
            pltpu.VMEM((16,), jnp.int32),
            pltpu.VMEM((16,), jnp.int32),
        ],
    )
    def sc_select(in_hbm, out_hbm,
                  key_v, sb_v, lb_v, cb_v, wb_v, out_v,
                  glmax_v, glchunk_v, red_k_v, red_i_v):
        cid = lax.axis_index("c")
        sid = lax.axis_index("s")

        @pl.when(sid == 0)
        def _():
            b = cid
            for k, dst in enumerate((key_v, sb_v, lb_v, cb_v, wb_v)):
                pltpu.sync_copy(in_hbm.at[pl.ds((b * 5 + k) * N_PAD, N_PAD)], dst)
            lane = lax.iota(jnp.int32, 16)
            for g in range(4 * K_PAD // 16):
                out_v[pl.ds(g * 16, 16)] = jnp.zeros((16,), jnp.int32)

            def build_group(g, carry):
                gm = jnp.full((16,), -2 ** 31, jnp.int32)
                gc = jnp.zeros((16,), jnp.int32)
                for c in range(16):
                    v = key_v[pl.ds((g * 16 + c) * 16, 16)]
                    upd = v > gm
                    gm = jnp.where(upd, v, gm)
                    gc = jnp.where(upd, g * 16 + c, gc)
                glmax_v[pl.ds(g * 16, 16)] = gm
                glchunk_v[pl.ds(g * 16, 16)] = gc
                return carry

            lax.fori_loop(0, NGRP, build_group, 0)

            def pick(t, carry):
                def gscan(g, mi_pair):
                    mv, mi = mi_pair
                    gm = glmax_v[pl.ds(g * 16, 16)]
                    gc = glchunk_v[pl.ds(g * 16, 16)]
                    idx = gc * 16 + lane
                    upd = gm > mv
                    return (jnp.where(upd, gm, mv), jnp.where(upd, idx, mi))

                mv, mi = lax.fori_loop(
                    0, NGRP, gscan,
                    (jnp.full((16,), -2 ** 31, jnp.int32), lane))
                for sh in (8, 4, 2, 1):
                    red_k_v[...] = mv
                    red_i_v[...] = mi
                    perm = lane ^ sh
                    k2 = plsc.load_gather(red_k_v, [perm])
                    i2 = plsc.load_gather(red_i_v, [perm])
                    better = (k2 > mv) | ((k2 == mv) & (i2 < mi))
                    mv = jnp.where(better, k2, mv)
                    mi = jnp.where(better, i2, mi)
                widx = mi[0]
                tmask = lane == 0
                plsc.store_scatter(key_v, [mi],
                                   jnp.full((16,), -2 ** 31, jnp.int32),
                                   mask=tmask)
                gstar = widx // 256
                gm = jnp.full((16,), -2 ** 31, jnp.int32)
                gc = jnp.zeros((16,), jnp.int32)
                for c in range(16):
                    v = key_v[pl.ds(gstar * 256 + c * 16, 16)]
                    upd = v > gm
                    gm = jnp.where(upd, v, gm)
                    gc = jnp.where(upd, gstar * 16 + c, gc)
                glmax_v[pl.ds(gstar * 16, 16)] = gm
                glchunk_v[pl.ds(gstar * 16, 16)] = gc
                tpos = jnp.full((16,), t, jnp.int32)
                for k, src in enumerate((sb_v, lb_v, cb_v, wb_v)):
                    plsc.store_scatter(out_v, [k * K_PAD + tpos],
                                       plsc.load_gather(src, [mi]), mask=tmask)
                return carry

            lax.fori_loop(0, TOPK, pick, 0)
            pltpu.sync_copy(out_v, out_hbm.at[pl.ds(b * 4 * K_PAD, 4 * K_PAD)])

    return sc_select


def _scores_labels(pred_logits):
    B = pred_logits.shape[0]
    logits = jnp.pad(pred_logits.reshape(B * N, C), ((0, 0), (0, C_PAD - C)),
                     constant_values=NEG)
    score_col, label_col = pl.pallas_call(
        _score_kernel,
        grid=(B * N // RB,),
        in_specs=[pl.BlockSpec((RB, C_PAD), lambda r: (r, 0))],
        out_specs=[pl.BlockSpec((RB, 1), lambda r: (r, 0)),
                   pl.BlockSpec((RB, 1), lambda r: (r, 0))],
        out_shape=[jax.ShapeDtypeStruct((B * N, 1), jnp.float32),
                   jax.ShapeDtypeStruct((B * N, 1), jnp.int32)],
    )(logits)
    return score_col.reshape(B, N), label_col.reshape(B, N)


def _nms_keys(c_p, w_p, s_p):
    B = c_p.shape[0]
    col = lambda x: x[..., None]
    row = lambda x: x[:, None, :]
    key = pl.pallas_call(
        _nms_kernel,
        grid=(B, N_PAD // TI),
        in_specs=[
            pl.BlockSpec((1, TI, 1), lambda b, i: (b, i, 0)),
            pl.BlockSpec((1, TI, 1), lambda b, i: (b, i, 0)),
            pl.BlockSpec((1, TI, 1), lambda b, i: (b, i, 0)),
            pl.BlockSpec((1, 1, N_PAD), lambda b, i: (b, 0, 0)),
            pl.BlockSpec((1, 1, N_PAD), lambda b, i: (b, 0, 0)),
            pl.BlockSpec((1, 1, N_PAD), lambda b, i: (b, 0, 0)),
        ],
        out_specs=pl.BlockSpec((1, TI, 1), lambda b, i: (b, i, 0)),
        out_shape=jax.ShapeDtypeStruct((B, N_PAD, 1), jnp.int32),
    )(col(c_p), col(w_p), col(s_p), row(c_p), row(w_p), row(s_p))
    return key.reshape(B, N_PAD)


def kernel(pred_logits, pred_segments, target_lengths):
    scores, labels = _scores_labels(pred_logits)
    seg = pred_segments * target_lengths[:, None, :]
    c = seg[..., 0]
    w = seg[..., 1]
    padr = lambda x, v: jnp.pad(x, ((0, 0), (0, N_PAD - N)), constant_values=v)
    c_p = padr(c, 0.0)
    w_p = padr(w, 0.0)
    s_p = padr(scores, NEG)
    key = _nms_keys(c_p, w_p, s_p)
    label_p = padr(labels, 0)
    bits = lambda x: lax.bitcast_convert_type(x, jnp.int32)
    packed = jnp.stack([key, bits(s_p), label_p, bits(c_p), bits(w_p)],
                       axis=1).reshape(-1)
    out = _make_sc_select()(packed)
    out = out.reshape(2, 4, K_PAD)[:, :, :TOPK]
    f32 = lambda x: lax.bitcast_convert_type(x, jnp.float32)
    out_scores = f32(out[:, 0])
    out_labels = out[:, 1]
    out_segments = jnp.stack([f32(out[:, 2]), f32(out[:, 3])], axis=-1)
    return out_scores, out_labels, out_segments
